# baseline (speedup 1.0000x reference)
import functools

import jax
import jax.numpy as jnp
from jax import lax
from jax.experimental import pallas as pl
from jax.experimental.pallas import tpu as pltpu
from jax.experimental.pallas import tpu_sc as plsc

N = 10000
E = 320000
D = 128

NC = 2
NS = 16
NW = NC * NS

EPT = E // NW
CW = 125
CH = EPT // CW
RING = 16
CPW = 104
NCP = 6
ROWS0 = CPW * NCP
TAIL = N - ROWS0 * NS
NR = N // 16

_MESH = plsc.VectorSubcoreMesh(core_axis_name="c", subcore_axis_name="s")


@functools.partial(
    pl.kernel,
    out_type=jax.ShapeDtypeStruct((NW, NR, 16), jnp.float32),
    mesh=_MESH,
    scratch_types=[
        pltpu.VMEM((EPT,), jnp.int32),
        pltpu.VMEM((NR, 16), jnp.float32),
    ],
    compiler_params=pltpu.CompilerParams(needs_layout_passes=False),
)
def _hist(row_hbm, parts_hbm, row_v, cnt_v):
    c = lax.axis_index("c")
    s = lax.axis_index("s")
    wid = s * NC + c
    base = pl.multiple_of(wid * EPT, 8)
    pltpu.sync_copy(row_hbm.at[pl.ds(base, EPT)], row_v)

    zeros16 = jnp.zeros((16,), jnp.float32)

    def zero_body(j, _):
        cnt_v[j] = zeros16
        return 0

    lax.fori_loop(0, NR, zero_body, 0)

    ones = jnp.ones((16,), jnp.float32)

    def body(j, _):
        idx = row_v[pl.ds(j * 16, 16)]
        hi = lax.shift_right_logical(idx, 4)
        lo = lax.bitwise_and(idx, 15)
        plsc.addupdate_scatter(cnt_v, [hi, lo], ones)
        return 0

    lax.fori_loop(0, EPT // 16, body, 0)
    pltpu.sync_copy(cnt_v, parts_hbm.at[wid])


def _dis_body(parts_ref, q_ref):
    cnt = jnp.sum(parts_ref[...], axis=0)
    q_ref[...] = jnp.where(cnt > 0, lax.rsqrt(jnp.where(cnt > 0, cnt, 1.0)),
                           0.0)


def _dis(parts):
    return pl.pallas_call(
        _dis_body,
        grid=(1,),
        in_specs=[pl.BlockSpec((NW, NR, 16), lambda i: (0, 0, 0))],
        out_specs=pl.BlockSpec((NR, 16), lambda i: (0, 0)),
        out_shape=jax.ShapeDtypeStruct((NR, 16), jnp.float32),
    )(parts)


def _mm_body(x_ref, w_ref, q_ref, out_ref):
    y = jnp.dot(x_ref[...], w_ref[...], preferred_element_type=jnp.float32)
    out_ref[...] = y * q_ref[...]


_MM_BN = 2000


def _mm(x, w, q):
    return pl.pallas_call(
        _mm_body,
        grid=(N // _MM_BN,),
        in_specs=[
            pl.BlockSpec((_MM_BN, D), lambda i: (i, 0)),
            pl.BlockSpec((D, D), lambda i: (0, 0)),
            pl.BlockSpec((_MM_BN, 1), lambda i: (i, 0)),
        ],
        out_specs=pl.BlockSpec((_MM_BN, D), lambda i: (i, 0)),
        out_shape=jax.ShapeDtypeStruct((N, D), jnp.float32),
    )(x, w, q)


@functools.partial(
    pl.kernel,
    out_type=jax.ShapeDtypeStruct((NC, N, D), jnp.float32),
    mesh=_MESH,
    scratch_types=[
        pltpu.VMEM((64, CW), jnp.int32),
        pltpu.VMEM((64, CW), jnp.int32),
        pltpu.VMEM((CW, D), jnp.float32),
        pltpu.VMEM((CW, D), jnp.float32),
        pltpu.VMEM_SHARED((N, D), jnp.float32),
        pltpu.SemaphoreType.DMA,
        pltpu.SemaphoreType.DMA,
        pltpu.SemaphoreType.DMA,
        pltpu.SemaphoreType.DMA,
        pltpu.SemaphoreType.DMA,
    ],
    compiler_params=pltpu.CompilerParams(needs_layout_passes=False),
)
def _prop(out_hbm, ei_hbm, zeros_hbm, agg_hbm,
          idx_row_v, idx_col_v, buf0, buf1, acc_sh,
          gsem0, gsem1, ssem0, ssem1, rsem):
    c = lax.axis_index("c")
    s = lax.axis_index("s")
    wid = s * NC + c
    base = pl.multiple_of(wid * CH, 8)

    pltpu.sync_copy(ei_hbm.at[0, pl.ds(base, 64)], idx_row_v)
    pltpu.sync_copy(ei_hbm.at[1, pl.ds(base, 64)], idx_col_v)

    zb = buf0.at[pl.ds(0, CPW)]
    pltpu.sync_copy(zeros_hbm, zb)
    for k in range(NCP):
        off = pl.multiple_of(s * ROWS0 + k * CPW, 8)
        pltpu.sync_copy(zb, acc_sh.at[pl.ds(off, CPW)])

    @pl.when(s == NS - 1)
    def _zero_tail():
        pltpu.sync_copy(buf0.at[pl.ds(0, TAIL)],
                        acc_sh.at[pl.ds(ROWS0 * NS, TAIL)])

    plsc.subcore_barrier()

    def gather(rj, buf, sem):
        return pltpu.async_copy(out_hbm.at[idx_col_v.at[rj]], buf, sem)

    def scatter(rj, buf, sem):
        return pltpu.async_copy(buf, acc_sh.at[idx_row_v.at[rj]], sem,
                                add=True)

    gather(0, buf0, gsem0)
    gather(1, buf1, gsem1)

    def body(j, _):
        c0 = 2 * j

        @pl.when(j == 16)
        def _issue_refill():
            src = pl.multiple_of(base + 64, 8)
            pltpu.async_copy(ei_hbm.at[0, pl.ds(src, 16)],
                             idx_row_v.at[pl.ds(0, 16)], rsem)
            pltpu.async_copy(ei_hbm.at[1, pl.ds(src, 16)],
                             idx_col_v.at[pl.ds(0, 16)], rsem)

        @pl.when(j == 30)
        def _drain_refill():
            pltpu.make_async_copy(ei_hbm.at[0, pl.ds(0, 16)],
                                  idx_row_v.at[pl.ds(0, 16)], rsem).wait()
            pltpu.make_async_copy(ei_hbm.at[1, pl.ds(0, 16)],
                                  idx_col_v.at[pl.ds(0, 16)], rsem).wait()

        r0 = lax.rem(c0, 64)
        r1 = lax.rem(c0 + 1, 64)
        rn0 = lax.rem(jnp.minimum(c0 + 2, CH - 1), 64)
        rn1 = lax.rem(jnp.minimum(c0 + 3, CH - 1), 64)
        pltpu.make_async_copy(out_hbm.at[idx_col_v.at[r0]], buf0, gsem0).wait()
        scatter(r0, buf0, ssem0)
        pltpu.make_async_copy(out_hbm.at[idx_col_v.at[r1]], buf1, gsem1).wait()
        scatter(r1, buf1, ssem1)
        pltpu.make_async_copy(buf0, acc_sh.at[idx_row_v.at[r0]], ssem0).wait()
        gather(rn0, buf0, gsem0)
        pltpu.make_async_copy(buf1, acc_sh.at[idx_row_v.at[r1]], ssem1).wait()
        gather(rn1, buf1, gsem1)
        return 0

    lax.fori_loop(0, CH // 2, body, 0)
    pltpu.make_async_copy(out_hbm.at[idx_col_v.at[0]], buf0, gsem0).wait()
    pltpu.make_async_copy(out_hbm.at[idx_col_v.at[0]], buf1, gsem1).wait()
    plsc.subcore_barrier()

    cb = buf0.at[pl.ds(0, CPW)]
    for k in range(NCP):
        off = pl.multiple_of(s * ROWS0 + k * CPW, 8)
        pltpu.sync_copy(acc_sh.at[pl.ds(off, CPW)], cb)
        pltpu.sync_copy(cb, agg_hbm.at[c, pl.ds(off, CPW)])

    @pl.when(s == NS - 1)
    def _copy_tail():
        tb = buf0.at[pl.ds(0, TAIL)]
        pltpu.sync_copy(acc_sh.at[pl.ds(ROWS0 * NS, TAIL)], tb)
        pltpu.sync_copy(tb, agg_hbm.at[c, pl.ds(ROWS0 * NS, TAIL)])


def _fin_body(p_ref, q_ref, b_ref, g_ref, be_ref, o_ref):
    a = (p_ref[0] + p_ref[1]) * q_ref[...] + b_ref[...]
    mu = jnp.mean(a, axis=1, keepdims=True)
    d = a - mu
    var = jnp.mean(d * d, axis=1, keepdims=True)
    h = d * lax.rsqrt(var + 1e-5) * g_ref[...] + be_ref[...]
    o_ref[...] = jnp.where(h > 0, h, 0.2 * h)


_FIN_BN = 2000


def _fin(p, q, b, g, be):
    return pl.pallas_call(
        _fin_body,
        grid=(N // _FIN_BN,),
        in_specs=[
            pl.BlockSpec((NC, _FIN_BN, D), lambda i: (0, i, 0)),
            pl.BlockSpec((_FIN_BN, 1), lambda i: (i, 0)),
            pl.BlockSpec((1, D), lambda i: (0, 0)),
            pl.BlockSpec((1, D), lambda i: (0, 0)),
            pl.BlockSpec((1, D), lambda i: (0, 0)),
        ],
        out_specs=pl.BlockSpec((_FIN_BN, D), lambda i: (i, 0)),
        out_shape=jax.ShapeDtypeStruct((N, D), jnp.float32),
    )(p, q, b, g, be)


def kernel(x, edge_index, W, adaptive_weight, bias, ln_gamma, ln_beta):
    del adaptive_weight
    parts = _hist(edge_index[0])
    ei_b, _ = lax.optimization_barrier((edge_index, parts))
    q = _dis(parts).reshape(N, 1)
    out_scaled = _mm(x, W, q)
    zeros = jnp.zeros((CPW, D), jnp.float32)
    ei3 = ei_b.reshape(2, E // CW, CW)
    aggp = _prop(out_scaled, ei3, zeros)
    return _fin(aggp, q, bias.reshape(1, D), ln_gamma.reshape(1, D),
                ln_beta.reshape(1, D))


# speedup vs baseline: 1.0351x; 1.0351x over previous
import functools

import jax
import jax.numpy as jnp
from jax import lax
from jax.experimental import pallas as pl
from jax.experimental.pallas import tpu as pltpu
from jax.experimental.pallas import tpu_sc as plsc

N = 10000
E = 320000
D = 128

NC = 2
NS = 16
NW = NC * NS

EPT = E // NW
CW = 125
CH = EPT // CW
RING = 16
CPW = 104
NCP = 6
ROWS0 = CPW * NCP
TAIL = N - ROWS0 * NS
NR = N // 16

_MESH = plsc.VectorSubcoreMesh(core_axis_name="c", subcore_axis_name="s")


@functools.partial(
    pl.kernel,
    out_type=jax.ShapeDtypeStruct((NW, NR, 16), jnp.float32),
    mesh=_MESH,
    scratch_types=[
        pltpu.VMEM((EPT,), jnp.int32),
        pltpu.VMEM((NR, 16), jnp.float32),
    ],
    compiler_params=pltpu.CompilerParams(needs_layout_passes=False),
)
def _hist(row_hbm, parts_hbm, row_v, cnt_v):
    c = lax.axis_index("c")
    s = lax.axis_index("s")
    wid = s * NC + c
    base = pl.multiple_of(wid * EPT, 8)
    pltpu.sync_copy(row_hbm.at[pl.ds(base, EPT)], row_v)

    zeros16 = jnp.zeros((16,), jnp.float32)

    def zero_body(j, _):
        cnt_v[j] = zeros16
        return 0

    lax.fori_loop(0, NR, zero_body, 0)

    ones = jnp.ones((16,), jnp.float32)

    def body(j, _):
        idx = row_v[pl.ds(j * 16, 16)]
        hi = lax.shift_right_logical(idx, 4)
        lo = lax.bitwise_and(idx, 15)
        plsc.addupdate_scatter(cnt_v, [hi, lo], ones)
        return 0

    lax.fori_loop(0, EPT // 16, body, 0)
    pltpu.sync_copy(cnt_v, parts_hbm.at[wid])


def _dis_body(parts_ref, q_ref):
    cnt = jnp.sum(parts_ref[...], axis=0)
    q_ref[...] = jnp.where(cnt > 0, lax.rsqrt(jnp.where(cnt > 0, cnt, 1.0)),
                           0.0)


def _dis(parts):
    return pl.pallas_call(
        _dis_body,
        grid=(1,),
        in_specs=[pl.BlockSpec((NW, NR, 16), lambda i: (0, 0, 0))],
        out_specs=pl.BlockSpec((NR, 16), lambda i: (0, 0)),
        out_shape=jax.ShapeDtypeStruct((NR, 16), jnp.float32),
    )(parts)


def _mm_body(x_ref, w_ref, q_ref, out_ref):
    y = jnp.dot(x_ref[...], w_ref[...], preferred_element_type=jnp.float32)
    out_ref[...] = y * q_ref[...]


_MM_BN = 2000


def _mm(x, w, q):
    return pl.pallas_call(
        _mm_body,
        grid=(N // _MM_BN,),
        in_specs=[
            pl.BlockSpec((_MM_BN, D), lambda i: (i, 0)),
            pl.BlockSpec((D, D), lambda i: (0, 0)),
            pl.BlockSpec((_MM_BN, 1), lambda i: (i, 0)),
        ],
        out_specs=pl.BlockSpec((_MM_BN, D), lambda i: (i, 0)),
        out_shape=jax.ShapeDtypeStruct((N, D), jnp.float32),
    )(x, w, q)


@functools.partial(
    pl.kernel,
    out_type=jax.ShapeDtypeStruct((NC, N, D), jnp.float32),
    mesh=_MESH,
    scratch_types=[
        pltpu.VMEM((64, CW), jnp.int32),
        pltpu.VMEM((64, CW), jnp.int32),
        pltpu.VMEM((CW, D), jnp.float32),
        pltpu.VMEM((CW, D), jnp.float32),
        pltpu.VMEM_SHARED((N, D), jnp.float32),
        pltpu.SemaphoreType.DMA,
        pltpu.SemaphoreType.DMA,
        pltpu.SemaphoreType.DMA,
        pltpu.SemaphoreType.DMA,
        pltpu.SemaphoreType.DMA,
    ],
    compiler_params=pltpu.CompilerParams(needs_layout_passes=False),
)
def _prop(out_hbm, ei_hbm, zeros_hbm, agg_hbm,
          idx_row_v, idx_col_v, buf0, buf1, acc_sh,
          gsem0, gsem1, ssem0, ssem1, rsem):
    c = lax.axis_index("c")
    s = lax.axis_index("s")
    wid = s * NC + c
    base = pl.multiple_of(wid * CH, 8)

    pltpu.sync_copy(ei_hbm.at[0, pl.ds(base, 64)], idx_row_v)
    pltpu.sync_copy(ei_hbm.at[1, pl.ds(base, 64)], idx_col_v)

    zb = buf0.at[pl.ds(0, CPW)]
    pltpu.sync_copy(zeros_hbm, zb)
    for k in range(NCP):
        off = pl.multiple_of(s * ROWS0 + k * CPW, 8)
        pltpu.sync_copy(zb, acc_sh.at[pl.ds(off, CPW)])

    @pl.when(s == NS - 1)
    def _zero_tail():
        pltpu.sync_copy(buf0.at[pl.ds(0, TAIL)],
                        acc_sh.at[pl.ds(ROWS0 * NS, TAIL)])

    plsc.subcore_barrier()

    def gather(rj, buf, sem):
        return pltpu.async_copy(out_hbm.at[idx_col_v.at[rj]], buf, sem)

    def scatter(rj, buf, sem):
        return pltpu.async_copy(buf, acc_sh.at[idx_row_v.at[rj]], sem,
                                add=True)

    gather(0, buf0, gsem0)
    gather(1, buf1, gsem1)

    def body(j, _):
        c0 = 2 * j

        @pl.when(j == 16)
        def _issue_refill():
            src = pl.multiple_of(base + 64, 8)
            pltpu.async_copy(ei_hbm.at[0, pl.ds(src, 16)],
                             idx_row_v.at[pl.ds(0, 16)], rsem)
            pltpu.async_copy(ei_hbm.at[1, pl.ds(src, 16)],
                             idx_col_v.at[pl.ds(0, 16)], rsem)

        @pl.when(j == 30)
        def _drain_refill():
            pltpu.make_async_copy(ei_hbm.at[0, pl.ds(0, 16)],
                                  idx_row_v.at[pl.ds(0, 16)], rsem).wait()
            pltpu.make_async_copy(ei_hbm.at[1, pl.ds(0, 16)],
                                  idx_col_v.at[pl.ds(0, 16)], rsem).wait()

        r0 = lax.rem(c0, 64)
        r1 = lax.rem(c0 + 1, 64)
        rn0 = lax.rem(jnp.minimum(c0 + 2, CH - 1), 64)
        rn1 = lax.rem(jnp.minimum(c0 + 3, CH - 1), 64)
        pltpu.make_async_copy(out_hbm.at[idx_col_v.at[r0]], buf0, gsem0).wait()
        scatter(r0, buf0, ssem0)
        pltpu.make_async_copy(out_hbm.at[idx_col_v.at[r1]], buf1, gsem1).wait()
        scatter(r1, buf1, ssem1)
        pltpu.make_async_copy(buf0, acc_sh.at[idx_row_v.at[r0]], ssem0).wait()
        gather(rn0, buf0, gsem0)
        pltpu.make_async_copy(buf1, acc_sh.at[idx_row_v.at[r1]], ssem1).wait()
        gather(rn1, buf1, gsem1)
        return 0

    lax.fori_loop(0, CH // 2, body, 0)
    pltpu.make_async_copy(out_hbm.at[idx_col_v.at[0]], buf0, gsem0).wait()
    pltpu.make_async_copy(out_hbm.at[idx_col_v.at[0]], buf1, gsem1).wait()
    plsc.subcore_barrier()

    cb = buf0.at[pl.ds(0, CPW)]
    for k in range(NCP):
        off = pl.multiple_of(s * ROWS0 + k * CPW, 8)
        pltpu.sync_copy(acc_sh.at[pl.ds(off, CPW)], cb)
        pltpu.sync_copy(cb, agg_hbm.at[c, pl.ds(off, CPW)])

    @pl.when(s == NS - 1)
    def _copy_tail():
        tb = buf0.at[pl.ds(0, TAIL)]
        pltpu.sync_copy(acc_sh.at[pl.ds(ROWS0 * NS, TAIL)], tb)
        pltpu.sync_copy(tb, agg_hbm.at[c, pl.ds(ROWS0 * NS, TAIL)])


def _fin_body(p_ref, q_ref, b_ref, g_ref, be_ref, o_ref):
    a = (p_ref[0] + p_ref[1]) * q_ref[...] + b_ref[...]
    mu = jnp.mean(a, axis=1, keepdims=True)
    d = a - mu
    var = jnp.mean(d * d, axis=1, keepdims=True)
    h = d * lax.rsqrt(var + 1e-5) * g_ref[...] + be_ref[...]
    o_ref[...] = jnp.where(h > 0, h, 0.2 * h)


_FIN_BN = 2000


def _fin(p, q, b, g, be):
    return pl.pallas_call(
        _fin_body,
        grid=(N // _FIN_BN,),
        in_specs=[
            pl.BlockSpec((NC, _FIN_BN, D), lambda i: (0, i, 0)),
            pl.BlockSpec((_FIN_BN, 1), lambda i: (i, 0)),
            pl.BlockSpec((1, D), lambda i: (0, 0)),
            pl.BlockSpec((1, D), lambda i: (0, 0)),
            pl.BlockSpec((1, D), lambda i: (0, 0)),
        ],
        out_specs=pl.BlockSpec((_FIN_BN, D), lambda i: (i, 0)),
        out_shape=jax.ShapeDtypeStruct((N, D), jnp.float32),
    )(p, q, b, g, be)


def kernel(x, edge_index, W, adaptive_weight, bias, ln_gamma, ln_beta):
    del adaptive_weight
    parts = _hist(edge_index[0])
    q = _dis(parts).reshape(N, 1)
    out_scaled = _mm(x, W, q)
    zeros = jnp.zeros((CPW, D), jnp.float32)
    ei3 = edge_index.reshape(2, E // CW, CW)
    aggp = _prop(out_scaled, ei3, zeros)
    return _fin(aggp, q, bias.reshape(1, D), ln_gamma.reshape(1, D),
                ln_beta.reshape(1, D))
